# baseline (speedup 1.0000x reference)
import jax
import jax.numpy as jnp
from jax import lax
from jax.experimental import pallas as pl
from jax.experimental.pallas import tpu as pltpu
from jax.experimental.pallas import tpu_sc as plsc

N_COMP_STATIC = 1000

NC = 2
NS = 16
NW = NC * NS
CH = 128
NBUF = 4
ZB = 32
CNT_W = 16


def _round_up(n, m):
    return ((n + m - 1) // m) * m


def _prep_indices(src, dst, s, s_pad):
    n = src.shape[0]
    n_pad = _round_up(n, NW * CH * NBUF)
    pad = n_pad - n
    if pad:
        src = jnp.concatenate([src, jnp.zeros((pad,), jnp.int32)])
        fill = s + (jnp.arange(pad, dtype=jnp.int32) % (s_pad - s))
        dst = jnp.concatenate([dst, fill])
    k = n_pad // (NW * CH)
    return src.reshape(NW, k, CH), dst.reshape(NW, k, CH)


def _fill_const(ref, rows, d, value):
    def body(i, carry):
        for j in range(d // 16):
            ref[i, pl.ds(j * 16, 16)] = jnp.full((16,), value, jnp.float32)
        return carry
    lax.fori_loop(0, rows, body, 0)


def _sc_mesh():
    return plsc.VectorSubcoreMesh(core_axis_name="c", subcore_axis_name="s",
                                  num_cores=NC, num_subcores=NS)


def _segsum_call(table, src3, dst3, s_pad):
    _, d = table.shape
    k = src3.shape[1]
    rpt = s_pad // NS
    kb = k // NBUF

    def body(table_h, src_h, dst_h, out_h, src_v, dst_v, zb_v, acc_sh,
             *bufs_and_sems):
        rows_v = bufs_and_sems[:1]
        gsems = bufs_and_sems[1:2]
        cid = lax.axis_index("c")
        sid = lax.axis_index("s")
        wid = sid * NC + cid
        base = sid * rpt
        _fill_const(zb_v, ZB, d, 0.0)

        def zacc(b, carry):
            pltpu.sync_copy(zb_v, acc_sh.at[pl.ds(base + b * ZB, ZB)])
            return carry
        lax.fori_loop(0, rpt // ZB, zacc, 0)
        pltpu.sync_copy(src_h.at[wid], src_v)
        pltpu.sync_copy(dst_h.at[wid], dst_v)
        plsc.subcore_barrier()

        def step(j, carry):
            pltpu.async_copy(table_h.at[src_v.at[j]], rows_v[0],
                             gsems[0]).wait()
            pltpu.sync_copy(rows_v[0], acc_sh.at[dst_v.at[j]], add=True)
            return carry
        lax.fori_loop(0, k, step, 0)
        plsc.subcore_barrier()
        pltpu.sync_copy(acc_sh.at[pl.ds(base, rpt)],
                        out_h.at[cid, pl.ds(base, rpt)])

    f = pl.kernel(
        body,
        out_type=jax.ShapeDtypeStruct((NC, s_pad, d), jnp.float32),
        mesh=_sc_mesh(),
        compiler_params=pltpu.CompilerParams(needs_layout_passes=False),
        scratch_types=[
            pltpu.VMEM((k, CH), jnp.int32),
            pltpu.VMEM((k, CH), jnp.int32),
            pltpu.VMEM((ZB, d), jnp.float32),
            pltpu.VMEM_SHARED((s_pad, d), jnp.float32),
        ] + [pltpu.VMEM((CH, d), jnp.float32)]
          + [pltpu.SemaphoreType.DMA],
    )
    return f(table, src3, dst3)


def _segcount_call(dst3, s_pad):
    k = dst3.shape[1]
    rows = s_pad // 128

    def body(dst_h, out_h, dst_v, cnt_v):
        cid = lax.axis_index("c")
        sid = lax.axis_index("s")
        wid = sid * NC + cid
        _fill_const(cnt_v, rows, 128, 0.0)
        pltpu.sync_copy(dst_h.at[wid], dst_v)
        ones = jnp.ones((16,), jnp.float32)

        def step(j, carry):
            def g_loop(g, c2):
                ii = dst_v[j, pl.ds(g * 16, 16)]
                row = lax.shift_right_logical(ii, 7)
                col = lax.bitwise_and(ii, 127)
                plsc.addupdate_scatter(cnt_v, [row, col], ones)
                return c2
            lax.fori_loop(0, CH // 16, g_loop, 0)
            return carry
        lax.fori_loop(0, k, step, 0)
        pltpu.sync_copy(cnt_v, out_h.at[wid])

    f = pl.kernel(
        body,
        out_type=jax.ShapeDtypeStruct((NW, rows, 128), jnp.float32),
        mesh=_sc_mesh(),
        compiler_params=pltpu.CompilerParams(needs_layout_passes=False),
        scratch_types=[
            pltpu.VMEM((k, CH), jnp.int32),
            pltpu.VMEM((rows, 128), jnp.float32),
        ],
    )
    return f(dst3)


def _cnt_reduce_call(parts):
    _, rows, d = parts.shape

    def body(a_ref, o_ref):
        o_ref[...] = jnp.sum(a_ref[...], axis=0)

    out = pl.pallas_call(
        body, out_shape=jax.ShapeDtypeStruct((rows, d), jnp.float32),
    )(parts)
    return out.reshape(rows * d, 1)


def _seq(x, dep):
    x, _ = lax.optimization_barrier((x, dep))
    return x


def _prelu(v, a):
    return jnp.where(v > 0, v, a * v)


def _mean(parts_ref, cnt_ref):
    s = parts_ref[0] + parts_ref[1]
    cnt = cnt_ref[...]
    return s / jnp.maximum(cnt, 1.0)


def _linear_call(h, w, b):
    m = h.shape[0]
    dout = w.shape[1]

    def body(h_ref, w_ref, b_ref, o_ref):
        o_ref[...] = jnp.dot(h_ref[...], w_ref[...],
                             preferred_element_type=jnp.float32) + b_ref[...]

    return pl.pallas_call(
        body, out_shape=jax.ShapeDtypeStruct((m, dout), jnp.float32),
    )(h, w, b.reshape(1, dout))


def _e_fusion_call(e_parts, cnt_parts, w2, b2, w3, b3, ae, n_e):
    d2 = w2.shape[1]
    d3 = w3.shape[1]

    def body(ep, cp, w2r, b2r, w3r, b3r, ae_r, e_o, ec_o, en_o):
        e = _prelu(_mean(ep, cp), ae_r[0, 0])
        e_o[...] = e[:n_e]
        ec_o[...] = (jnp.dot(e, w2r[...], preferred_element_type=jnp.float32)
                     + b2r[...])[:n_e]
        en_o[...] = (jnp.dot(e, w3r[...], preferred_element_type=jnp.float32)
                     + b3r[...])[:n_e]

    return pl.pallas_call(
        body,
        out_shape=[
            jax.ShapeDtypeStruct((n_e, e_parts.shape[2]), jnp.float32),
            jax.ShapeDtypeStruct((n_e, d2), jnp.float32),
            jax.ShapeDtypeStruct((n_e, d3), jnp.float32),
        ],
    )(e_parts, cnt_parts, w2, b2.reshape(1, d2), w3, b3.reshape(1, d3),
      ae.reshape(1, 1))


def _c_fusion_call(c_parts, cnt_parts, w4, b4, ac, n_c):
    din = w4.shape[0]
    d4 = w4.shape[1]

    def body(cparts, cnt, w4r, b4r, ac_r, c_o, cn_o):
        c = _prelu(_mean(cparts, cnt), ac_r[0, 0])[:, :din]
        c_o[...] = c[:n_c]
        cn_o[...] = (jnp.dot(c, w4r[...], preferred_element_type=jnp.float32)
                     + b4r[...])[:n_c]

    return pl.pallas_call(
        body,
        out_shape=[
            jax.ShapeDtypeStruct((n_c, din), jnp.float32),
            jax.ShapeDtypeStruct((n_c, d4), jnp.float32),
        ],
    )(c_parts, cnt_parts, w4, b4.reshape(1, d4), ac.reshape(1, 1))


def _n_fusion_call(nfe_parts, cnfe, nfc_parts, cnfc, an, alpha, n_n):
    d = nfe_parts.shape[2]

    def body(ep, ec, cp, cc, an_r, al_r, h_o):
        n = _mean(ep, ec) + _mean(cp, cc)
        n = _prelu(n, an_r[0, 0])
        h = _prelu(n, al_r[0, 0])
        h_o[...] = h[:n_n]

    return pl.pallas_call(
        body,
        out_shape=jax.ShapeDtypeStruct((n_n, d), jnp.float32),
    )(nfe_parts, cnfe, nfc_parts, cnfc, an.reshape(1, 1), alpha.reshape(1, 1))


def kernel(x, hyperedge_index, hyperedge_component_index, node_component_index,
           num_nodes, num_edges, num_components, params, alpha_act):
    n_n = x.shape[0]
    n_e = hyperedge_component_index.shape[1]
    n_c = N_COMP_STATIC

    s_e = _round_up(n_e + 1, NS * ZB)
    s_c = _round_up(n_c + 1, NS * ZB)
    s_n = _round_up(n_n + 1, NS * ZB)

    hei = hyperedge_index
    hci = hyperedge_component_index
    nci = node_component_index

    se_src, se_dst = _prep_indices(hei[0], hei[1], n_e, s_e)
    sn_src, sn_dst = _prep_indices(hei[1], hei[0], n_n, s_n)
    sc_src, sc_dst = _prep_indices(hci[0], hci[1], n_c, s_c)
    ncs_src, ncs_dst = _prep_indices(nci[1], nci[0], n_n, s_n)

    cnt_e = _cnt_reduce_call(_segcount_call(se_dst, s_e))
    cnt_ne = _cnt_reduce_call(_segcount_call(sn_dst, s_n))
    cnt_c = _cnt_reduce_call(_segcount_call(sc_dst, s_c))
    cnt_nc = _cnt_reduce_call(_segcount_call(ncs_dst, s_n))

    h = x
    e = c = None
    for p in params:
        w2p = jnp.pad(p['W2'], ((0, 0), (0, 128 - p['W2'].shape[1])))
        b2p = jnp.pad(p['b2'], (0, 128 - p['b2'].shape[0]))
        xe = _linear_call(h, p['W1'], p['b1'])
        e_parts = _segsum_call(xe, se_src, se_dst, s_e)
        e, ec, en = _e_fusion_call(e_parts, cnt_e, w2p, b2p,
                                   p['W3'], p['b3'], p['ae'], n_e)
        c_parts = _segsum_call(ec, sc_src, sc_dst, s_c)
        c, cn = _c_fusion_call(c_parts, cnt_c, p['W4'], p['b4'], p['ac'], n_c)
        nfe_parts = _segsum_call(_seq(en, c_parts), sn_src, sn_dst, s_n)
        nfc_parts = _segsum_call(_seq(cn, nfe_parts), ncs_src, ncs_dst, s_n)
        h = _n_fusion_call(nfe_parts, cnt_ne, nfc_parts, cnt_nc,
                           p['an'], alpha_act, n_n)
    return (h, e, c)


# speedup vs baseline: 1.0297x; 1.0297x over previous
import jax
import jax.numpy as jnp
from jax import lax
from jax.experimental import pallas as pl
from jax.experimental.pallas import tpu as pltpu
from jax.experimental.pallas import tpu_sc as plsc

N_COMP_STATIC = 1000

NC = 2
NS = 16
NW = NC * NS
CH = 128
NBUF = 4
ZB = 32
CNT_W = 16


def _round_up(n, m):
    return ((n + m - 1) // m) * m


def _prep_indices(src, dst, s, s_pad):
    n = src.shape[0]
    n_pad = _round_up(n, NW * CH * NBUF)
    pad = n_pad - n
    if pad:
        src = jnp.concatenate([src, jnp.zeros((pad,), jnp.int32)])
        fill = s + (jnp.arange(pad, dtype=jnp.int32) % (s_pad - s))
        dst = jnp.concatenate([dst, fill])
    k = n_pad // (NW * CH)
    return src.reshape(NW, k, CH), dst.reshape(NW, k, CH)


def _fill_const(ref, rows, d, value):
    def body(i, carry):
        for j in range(d // 16):
            ref[i, pl.ds(j * 16, 16)] = jnp.full((16,), value, jnp.float32)
        return carry
    lax.fori_loop(0, rows, body, 0)


def _sc_mesh():
    return plsc.VectorSubcoreMesh(core_axis_name="c", subcore_axis_name="s",
                                  num_cores=NC, num_subcores=NS)


def _segsum_call(table, src3, dst3, s_pad):
    _, d = table.shape
    k = src3.shape[1]
    rpt = s_pad // NS
    kb = k // NBUF

    def body(table_h, src_h, dst_h, out_h, src_v, dst_v, zb_v, acc_sh,
             *bufs_and_sems):
        rows_v = bufs_and_sems[:1]
        gsems = bufs_and_sems[1:2]
        cid = lax.axis_index("c")
        sid = lax.axis_index("s")
        wid = sid * NC + cid
        base = sid * rpt
        _fill_const(zb_v, ZB, d, 0.0)

        def zacc(b, carry):
            pltpu.sync_copy(zb_v, acc_sh.at[pl.ds(base + b * ZB, ZB)])
            return carry
        lax.fori_loop(0, rpt // ZB, zacc, 0)
        pltpu.sync_copy(src_h.at[wid], src_v)
        pltpu.sync_copy(dst_h.at[wid], dst_v)
        plsc.subcore_barrier()

        def step(j, carry):
            pltpu.async_copy(table_h.at[src_v.at[j]], rows_v[0],
                             gsems[0]).wait()
            pltpu.sync_copy(rows_v[0], acc_sh.at[dst_v.at[j]], add=True)
            return carry
        lax.fori_loop(0, k, step, 0)
        plsc.subcore_barrier()
        pltpu.sync_copy(acc_sh.at[pl.ds(base, rpt)],
                        out_h.at[cid, pl.ds(base, rpt)])

    f = pl.kernel(
        body,
        out_type=jax.ShapeDtypeStruct((NC, s_pad, d), jnp.float32),
        mesh=_sc_mesh(),
        compiler_params=pltpu.CompilerParams(needs_layout_passes=False),
        scratch_types=[
            pltpu.VMEM((k, CH), jnp.int32),
            pltpu.VMEM((k, CH), jnp.int32),
            pltpu.VMEM((ZB, d), jnp.float32),
            pltpu.VMEM_SHARED((s_pad, d), jnp.float32),
        ] + [pltpu.VMEM((CH, d), jnp.float32)]
          + [pltpu.SemaphoreType.DMA],
    )
    return f(table, src3, dst3)


def _segcount_call(dst3, s_pad):
    k = dst3.shape[1]
    rows = s_pad // 128

    def body(dst_h, out_h, dst_v, cnt_v):
        cid = lax.axis_index("c")
        sid = lax.axis_index("s")
        wid = sid * NC + cid
        _fill_const(cnt_v, rows, 128, 0.0)
        pltpu.sync_copy(dst_h.at[wid], dst_v)
        ones = jnp.ones((16,), jnp.float32)

        def step(j, carry):
            def g_loop(g, c2):
                ii = dst_v[j, pl.ds(g * 16, 16)]
                row = lax.shift_right_logical(ii, 7)
                col = lax.bitwise_and(ii, 127)
                plsc.addupdate_scatter(cnt_v, [row, col], ones)
                return c2
            lax.fori_loop(0, CH // 16, g_loop, 0)
            return carry
        lax.fori_loop(0, k, step, 0)
        pltpu.sync_copy(cnt_v, out_h.at[wid])

    f = pl.kernel(
        body,
        out_type=jax.ShapeDtypeStruct((NW, rows, 128), jnp.float32),
        mesh=_sc_mesh(),
        compiler_params=pltpu.CompilerParams(needs_layout_passes=False),
        scratch_types=[
            pltpu.VMEM((k, CH), jnp.int32),
            pltpu.VMEM((rows, 128), jnp.float32),
        ],
    )
    return f(dst3)


def _cnt_reduce_call(parts):
    _, rows, d = parts.shape

    def body(a_ref, o_ref):
        o_ref[...] = jnp.sum(a_ref[...], axis=0)

    out = pl.pallas_call(
        body, out_shape=jax.ShapeDtypeStruct((rows, d), jnp.float32),
    )(parts)
    return out.reshape(rows * d, 1)


def _seq(x, dep):
    x, _ = lax.optimization_barrier((x, dep))
    return x


def _prelu(v, a):
    return jnp.where(v > 0, v, a * v)


def _mean(parts_ref, cnt_ref):
    s = parts_ref[0] + parts_ref[1]
    cnt = cnt_ref[...]
    return s / jnp.maximum(cnt, 1.0)


def _linear_call(h, w, b):
    m = h.shape[0]
    dout = w.shape[1]

    def body(h_ref, w_ref, b_ref, o_ref):
        o_ref[...] = jnp.dot(h_ref[...], w_ref[...],
                             preferred_element_type=jnp.float32) + b_ref[...]

    return pl.pallas_call(
        body, out_shape=jax.ShapeDtypeStruct((m, dout), jnp.float32),
    )(h, w, b.reshape(1, dout))


def _e_fusion_call(e_parts, cnt_parts, w2, b2, w3, b3, ae, n_e):
    d2 = w2.shape[1]
    d3 = w3.shape[1]

    def body(ep, cp, w2r, b2r, w3r, b3r, ae_r, e_o, ec_o, en_o):
        e = _prelu(_mean(ep, cp), ae_r[0, 0])
        e_o[...] = e[:n_e]
        ec_o[...] = (jnp.dot(e, w2r[...], preferred_element_type=jnp.float32)
                     + b2r[...])[:n_e]
        en_o[...] = (jnp.dot(e, w3r[...], preferred_element_type=jnp.float32)
                     + b3r[...])[:n_e]

    return pl.pallas_call(
        body,
        out_shape=[
            jax.ShapeDtypeStruct((n_e, e_parts.shape[2]), jnp.float32),
            jax.ShapeDtypeStruct((n_e, d2), jnp.float32),
            jax.ShapeDtypeStruct((n_e, d3), jnp.float32),
        ],
    )(e_parts, cnt_parts, w2, b2.reshape(1, d2), w3, b3.reshape(1, d3),
      ae.reshape(1, 1))


def _c_fusion_call(c_parts, cnt_parts, w4, b4, ac, n_c):
    din = w4.shape[0]
    d4 = w4.shape[1]

    def body(cparts, cnt, w4r, b4r, ac_r, c_o, cn_o):
        c = _prelu(_mean(cparts, cnt), ac_r[0, 0])[:, :din]
        c_o[...] = c[:n_c]
        cn_o[...] = (jnp.dot(c, w4r[...], preferred_element_type=jnp.float32)
                     + b4r[...])[:n_c]

    return pl.pallas_call(
        body,
        out_shape=[
            jax.ShapeDtypeStruct((n_c, din), jnp.float32),
            jax.ShapeDtypeStruct((n_c, d4), jnp.float32),
        ],
    )(c_parts, cnt_parts, w4, b4.reshape(1, d4), ac.reshape(1, 1))


def _n_fusion_call(nfe_parts, cnfe, nfc_parts, cnfc, an, alpha, n_n):
    d = nfe_parts.shape[2]

    def body(ep, ec, cp, cc, an_r, al_r, h_o):
        n = _mean(ep, ec) + _mean(cp, cc)
        n = _prelu(n, an_r[0, 0])
        h = _prelu(n, al_r[0, 0])
        h_o[...] = h[:n_n]

    return pl.pallas_call(
        body,
        out_shape=jax.ShapeDtypeStruct((n_n, d), jnp.float32),
    )(nfe_parts, cnfe, nfc_parts, cnfc, an.reshape(1, 1), alpha.reshape(1, 1))


def kernel(x, hyperedge_index, hyperedge_component_index, node_component_index,
           num_nodes, num_edges, num_components, params, alpha_act):
    n_n = x.shape[0]
    n_e = hyperedge_component_index.shape[1]
    n_c = N_COMP_STATIC

    s_e = _round_up(n_e + 1, NS * ZB)
    s_c = _round_up(n_c + 1, NS * ZB)
    s_n = _round_up(n_n + 1, NS * ZB)

    hei = hyperedge_index
    hci = hyperedge_component_index
    nci = node_component_index

    se_src, se_dst = _prep_indices(hei[0], hei[1], n_e, s_e)
    sn_src, sn_dst = _prep_indices(hei[1], hei[0], n_n, s_n)
    sc_src, sc_dst = _prep_indices(hci[0], hci[1], n_c, s_c)
    ncs_src, ncs_dst = _prep_indices(nci[1], nci[0], n_n, s_n)

    cnt_e = _cnt_reduce_call(_segcount_call(se_dst, s_e))
    cnt_ne = _cnt_reduce_call(_segcount_call(sn_dst, s_n))
    cnt_c = _cnt_reduce_call(_segcount_call(sc_dst, s_c))
    cnt_nc = _cnt_reduce_call(_segcount_call(ncs_dst, s_n))

    h = x
    e = c = None
    for p in params:
        w2p = jnp.pad(p['W2'], ((0, 0), (0, 128 - p['W2'].shape[1])))
        b2p = jnp.pad(p['b2'], (0, 128 - p['b2'].shape[0]))
        xe = _linear_call(h, p['W1'], p['b1'])
        e_parts = _segsum_call(xe, se_src, se_dst, s_e)
        e, ec, en = _e_fusion_call(e_parts, cnt_e, w2p, b2p,
                                   p['W3'], p['b3'], p['ae'], n_e)
        c_parts = _segsum_call(ec, sc_src, sc_dst, s_c)
        c, cn = _c_fusion_call(c_parts, cnt_c, p['W4'], p['b4'], p['ac'], n_c)
        nfe_parts = _segsum_call(en, sn_src, sn_dst, s_n)
        nfc_parts = _segsum_call(cn, ncs_src, ncs_dst, s_n)
        h = _n_fusion_call(nfe_parts, cnt_ne, nfc_parts, cnt_nc,
                           p['an'], alpha_act, n_n)
    return (h, e, c)


# speedup vs baseline: 1.0297x; 1.0000x over previous
import jax
import jax.numpy as jnp
from jax import lax
from jax.experimental import pallas as pl
from jax.experimental.pallas import tpu as pltpu
from jax.experimental.pallas import tpu_sc as plsc

N_COMP_STATIC = 1000

NC = 2
NS = 16
NW = NC * NS
CH = 128
NBUF = 4
ZB = 32
CNT_W = 16


def _round_up(n, m):
    return ((n + m - 1) // m) * m


def _prep_indices(src, dst, s, s_pad):
    n = src.shape[0]
    n_pad = _round_up(n, NW * CH * NBUF)
    pad = n_pad - n
    if pad:
        src = jnp.concatenate([src, jnp.zeros((pad,), jnp.int32)])
        fill = s + (jnp.arange(pad, dtype=jnp.int32) % (s_pad - s))
        dst = jnp.concatenate([dst, fill])
    k = n_pad // (NW * CH)
    return src.reshape(NW, k, CH), dst.reshape(NW, k, CH)


def _fill_const(ref, rows, d, value):
    def body(i, carry):
        for j in range(d // 16):
            ref[i, pl.ds(j * 16, 16)] = jnp.full((16,), value, jnp.float32)
        return carry
    lax.fori_loop(0, rows, body, 0)


def _sc_mesh():
    return plsc.VectorSubcoreMesh(core_axis_name="c", subcore_axis_name="s",
                                  num_cores=NC, num_subcores=NS)


def _segsum_call(table, src3, dst3, s_pad):
    _, d = table.shape
    k = src3.shape[1]
    rpt = s_pad // NS
    kb = k // NBUF

    def body(table_h, src_h, dst_h, out_h, src_v, dst_v, zb_v, acc_sh,
             *bufs_and_sems):
        rows_v = bufs_and_sems[:1]
        gsems = bufs_and_sems[1:2]
        cid = lax.axis_index("c")
        sid = lax.axis_index("s")
        wid = sid * NC + cid
        base = sid * rpt
        _fill_const(zb_v, ZB, d, 0.0)

        def zacc(b, carry):
            pltpu.sync_copy(zb_v, acc_sh.at[pl.ds(base + b * ZB, ZB)])
            return carry
        lax.fori_loop(0, rpt // ZB, zacc, 0)
        pltpu.sync_copy(src_h.at[wid], src_v)
        pltpu.sync_copy(dst_h.at[wid], dst_v)
        plsc.subcore_barrier()

        def step(j, carry):
            pltpu.async_copy(table_h.at[src_v.at[j]], rows_v[0],
                             gsems[0]).wait()
            pltpu.sync_copy(rows_v[0], acc_sh.at[dst_v.at[j]], add=True)
            return carry
        lax.fori_loop(0, k, step, 0)
        plsc.subcore_barrier()
        pltpu.sync_copy(acc_sh.at[pl.ds(base, rpt)],
                        out_h.at[cid, pl.ds(base, rpt)])

    f = pl.kernel(
        body,
        out_type=jax.ShapeDtypeStruct((NC, s_pad, d), jnp.float32),
        mesh=_sc_mesh(),
        name=f"segsum_s{s_pad}_k{k}",
        compiler_params=pltpu.CompilerParams(needs_layout_passes=False),
        scratch_types=[
            pltpu.VMEM((k, CH), jnp.int32),
            pltpu.VMEM((k, CH), jnp.int32),
            pltpu.VMEM((ZB, d), jnp.float32),
            pltpu.VMEM_SHARED((s_pad, d), jnp.float32),
        ] + [pltpu.VMEM((CH, d), jnp.float32)]
          + [pltpu.SemaphoreType.DMA],
    )
    return f(table, src3, dst3)


def _segcount_call(dst3, s_pad):
    k = dst3.shape[1]
    rows = s_pad // 128

    def body(dst_h, out_h, dst_v, cnt_v):
        cid = lax.axis_index("c")
        sid = lax.axis_index("s")
        wid = sid * NC + cid
        _fill_const(cnt_v, rows, 128, 0.0)
        pltpu.sync_copy(dst_h.at[wid], dst_v)
        ones = jnp.ones((16,), jnp.float32)

        def step(j, carry):
            def g_loop(g, c2):
                ii = dst_v[j, pl.ds(g * 16, 16)]
                row = lax.shift_right_logical(ii, 7)
                col = lax.bitwise_and(ii, 127)
                plsc.addupdate_scatter(cnt_v, [row, col], ones)
                return c2
            lax.fori_loop(0, CH // 16, g_loop, 0)
            return carry
        lax.fori_loop(0, k, step, 0)
        pltpu.sync_copy(cnt_v, out_h.at[wid])

    f = pl.kernel(
        body,
        out_type=jax.ShapeDtypeStruct((NW, rows, 128), jnp.float32),
        mesh=_sc_mesh(),
        name=f"segcount_s{s_pad}_k{k}",
        compiler_params=pltpu.CompilerParams(needs_layout_passes=False),
        scratch_types=[
            pltpu.VMEM((k, CH), jnp.int32),
            pltpu.VMEM((rows, 128), jnp.float32),
        ],
    )
    return f(dst3)


def _cnt_reduce_call(parts):
    _, rows, d = parts.shape

    def body(a_ref, o_ref):
        o_ref[...] = jnp.sum(a_ref[...], axis=0)

    out = pl.pallas_call(
        body, out_shape=jax.ShapeDtypeStruct((rows, d), jnp.float32),
    )(parts)
    return out.reshape(rows * d, 1)


def _seq(x, dep):
    x, _ = lax.optimization_barrier((x, dep))
    return x


def _prelu(v, a):
    return jnp.where(v > 0, v, a * v)


def _mean(parts_ref, cnt_ref):
    s = parts_ref[0] + parts_ref[1]
    cnt = cnt_ref[...]
    return s / jnp.maximum(cnt, 1.0)


def _linear_call(h, w, b):
    m = h.shape[0]
    dout = w.shape[1]

    def body(h_ref, w_ref, b_ref, o_ref):
        o_ref[...] = jnp.dot(h_ref[...], w_ref[...],
                             preferred_element_type=jnp.float32) + b_ref[...]

    return pl.pallas_call(
        body, out_shape=jax.ShapeDtypeStruct((m, dout), jnp.float32),
    )(h, w, b.reshape(1, dout))


def _e_fusion_call(e_parts, cnt_parts, w2, b2, w3, b3, ae, n_e):
    d2 = w2.shape[1]
    d3 = w3.shape[1]

    def body(ep, cp, w2r, b2r, w3r, b3r, ae_r, e_o, ec_o, en_o):
        e = _prelu(_mean(ep, cp), ae_r[0, 0])
        e_o[...] = e[:n_e]
        ec_o[...] = (jnp.dot(e, w2r[...], preferred_element_type=jnp.float32)
                     + b2r[...])[:n_e]
        en_o[...] = (jnp.dot(e, w3r[...], preferred_element_type=jnp.float32)
                     + b3r[...])[:n_e]

    return pl.pallas_call(
        body,
        out_shape=[
            jax.ShapeDtypeStruct((n_e, e_parts.shape[2]), jnp.float32),
            jax.ShapeDtypeStruct((n_e, d2), jnp.float32),
            jax.ShapeDtypeStruct((n_e, d3), jnp.float32),
        ],
    )(e_parts, cnt_parts, w2, b2.reshape(1, d2), w3, b3.reshape(1, d3),
      ae.reshape(1, 1))


def _c_fusion_call(c_parts, cnt_parts, w4, b4, ac, n_c):
    din = w4.shape[0]
    d4 = w4.shape[1]

    def body(cparts, cnt, w4r, b4r, ac_r, c_o, cn_o):
        c = _prelu(_mean(cparts, cnt), ac_r[0, 0])[:, :din]
        c_o[...] = c[:n_c]
        cn_o[...] = (jnp.dot(c, w4r[...], preferred_element_type=jnp.float32)
                     + b4r[...])[:n_c]

    return pl.pallas_call(
        body,
        out_shape=[
            jax.ShapeDtypeStruct((n_c, din), jnp.float32),
            jax.ShapeDtypeStruct((n_c, d4), jnp.float32),
        ],
    )(c_parts, cnt_parts, w4, b4.reshape(1, d4), ac.reshape(1, 1))


def _n_fusion_call(nfe_parts, cnfe, nfc_parts, cnfc, an, alpha, n_n):
    d = nfe_parts.shape[2]

    def body(ep, ec, cp, cc, an_r, al_r, h_o):
        n = _mean(ep, ec) + _mean(cp, cc)
        n = _prelu(n, an_r[0, 0])
        h = _prelu(n, al_r[0, 0])
        h_o[...] = h[:n_n]

    return pl.pallas_call(
        body,
        out_shape=jax.ShapeDtypeStruct((n_n, d), jnp.float32),
    )(nfe_parts, cnfe, nfc_parts, cnfc, an.reshape(1, 1), alpha.reshape(1, 1))


def kernel(x, hyperedge_index, hyperedge_component_index, node_component_index,
           num_nodes, num_edges, num_components, params, alpha_act):
    n_n = x.shape[0]
    n_e = hyperedge_component_index.shape[1]
    n_c = N_COMP_STATIC

    s_e = _round_up(n_e + 1, NS * ZB)
    s_c = _round_up(n_c + 1, NS * ZB)
    s_n = _round_up(n_n + 1, NS * ZB)

    hei = hyperedge_index
    hci = hyperedge_component_index
    nci = node_component_index

    se_src, se_dst = _prep_indices(hei[0], hei[1], n_e, s_e)
    sn_src, sn_dst = _prep_indices(hei[1], hei[0], n_n, s_n)
    sc_src, sc_dst = _prep_indices(hci[0], hci[1], n_c, s_c)
    ncs_src, ncs_dst = _prep_indices(nci[1], nci[0], n_n, s_n)

    cnt_e = _cnt_reduce_call(_segcount_call(se_dst, s_e))
    cnt_ne = _cnt_reduce_call(_segcount_call(sn_dst, s_n))
    cnt_c = _cnt_reduce_call(_segcount_call(sc_dst, s_c))
    cnt_nc = _cnt_reduce_call(_segcount_call(ncs_dst, s_n))

    h = x
    e = c = None
    for p in params:
        w2p = jnp.pad(p['W2'], ((0, 0), (0, 128 - p['W2'].shape[1])))
        b2p = jnp.pad(p['b2'], (0, 128 - p['b2'].shape[0]))
        xe = _linear_call(h, p['W1'], p['b1'])
        e_parts = _segsum_call(xe, se_src, se_dst, s_e)
        e, ec, en = _e_fusion_call(e_parts, cnt_e, w2p, b2p,
                                   p['W3'], p['b3'], p['ae'], n_e)
        c_parts = _segsum_call(ec, sc_src, sc_dst, s_c)
        c, cn = _c_fusion_call(c_parts, cnt_c, p['W4'], p['b4'], p['ac'], n_c)
        nfe_parts = _segsum_call(en, sn_src, sn_dst, s_n)
        nfc_parts = _segsum_call(cn, ncs_src, ncs_dst, s_n)
        h = _n_fusion_call(nfe_parts, cnt_ne, nfc_parts, cnt_nc,
                           p['an'], alpha_act, n_n)
    return (h, e, c)


# speedup vs baseline: 4.1517x; 4.0319x over previous
import jax
import jax.numpy as jnp
from jax import lax
from jax.experimental import pallas as pl
from jax.experimental.pallas import tpu as pltpu
from jax.experimental.pallas import tpu_sc as plsc

N_COMP_STATIC = 1000

NC = 2
NS = 16
NW = NC * NS
CH = 128
NBUF = 4
ZB = 32
CNT_W = 16


def _round_up(n, m):
    return ((n + m - 1) // m) * m


def _prep_indices(src, dst, s, s_pad, r):
    n = src.shape[0]
    n_pad = _round_up(n, NW * CH)
    pad = n_pad - n
    if pad:
        ar = jnp.arange(pad, dtype=jnp.int32)
        src = jnp.concatenate([src, ar % r])
        dst = jnp.concatenate([dst, s + ar % (s_pad - s)])
    k = n_pad // (NW * CH)
    return src.reshape(NW, k, CH), dst.reshape(NW, k, CH)


def _fill_const(ref, rows, d, value):
    def body(i, carry):
        for j in range(d // 16):
            ref[i, pl.ds(j * 16, 16)] = jnp.full((16,), value, jnp.float32)
        return carry
    lax.fori_loop(0, rows, body, 0)


def _sc_mesh():
    return plsc.VectorSubcoreMesh(core_axis_name="c", subcore_axis_name="s",
                                  num_cores=NC, num_subcores=NS)


def _segsum_call(table, src3, dst3, s_pad):
    _, d = table.shape
    k = src3.shape[1]
    rpt = s_pad // NS
    kb = k // NBUF

    def body(table_h, src_h, dst_h, out_h, src_v, dst_v, zb_v, acc_sh,
             *bufs_and_sems):
        rows_v = bufs_and_sems[:1]
        gsems = bufs_and_sems[1:2]
        cid = lax.axis_index("c")
        sid = lax.axis_index("s")
        wid = sid * NC + cid
        base = sid * rpt
        _fill_const(zb_v, ZB, d, 0.0)

        def zacc(b, carry):
            pltpu.sync_copy(zb_v, acc_sh.at[pl.ds(base + b * ZB, ZB)])
            return carry
        lax.fori_loop(0, rpt // ZB, zacc, 0)
        pltpu.sync_copy(src_h.at[wid], src_v)
        pltpu.sync_copy(dst_h.at[wid], dst_v)
        plsc.subcore_barrier()

        def step(j, carry):
            pltpu.async_copy(table_h.at[src_v.at[j]], rows_v[0],
                             gsems[0]).wait()
            pltpu.sync_copy(rows_v[0], acc_sh.at[dst_v.at[j]], add=True)
            return carry
        lax.fori_loop(0, k, step, 0)
        plsc.subcore_barrier()
        pltpu.sync_copy(acc_sh.at[pl.ds(base, rpt)],
                        out_h.at[cid, pl.ds(base, rpt)])

    f = pl.kernel(
        body,
        out_type=jax.ShapeDtypeStruct((NC, s_pad, d), jnp.float32),
        mesh=_sc_mesh(),
        name=f"segsum_s{s_pad}_k{k}",
        compiler_params=pltpu.CompilerParams(needs_layout_passes=False),
        scratch_types=[
            pltpu.VMEM((k, CH), jnp.int32),
            pltpu.VMEM((k, CH), jnp.int32),
            pltpu.VMEM((ZB, d), jnp.float32),
            pltpu.VMEM_SHARED((s_pad, d), jnp.float32),
        ] + [pltpu.VMEM((CH, d), jnp.float32)]
          + [pltpu.SemaphoreType.DMA],
    )
    return f(table, src3, dst3)


def _segcount_call(dst3, s_pad):
    k = dst3.shape[1]
    rows = s_pad // 128

    def body(dst_h, out_h, dst_v, cnt_v):
        cid = lax.axis_index("c")
        sid = lax.axis_index("s")
        wid = sid * NC + cid
        _fill_const(cnt_v, rows, 128, 0.0)
        pltpu.sync_copy(dst_h.at[wid], dst_v)
        ones = jnp.ones((16,), jnp.float32)

        def step(j, carry):
            def g_loop(g, c2):
                ii = dst_v[j, pl.ds(g * 16, 16)]
                row = lax.shift_right_logical(ii, 7)
                col = lax.bitwise_and(ii, 127)
                plsc.addupdate_scatter(cnt_v, [row, col], ones)
                return c2
            lax.fori_loop(0, CH // 16, g_loop, 0)
            return carry
        lax.fori_loop(0, k, step, 0)
        pltpu.sync_copy(cnt_v, out_h.at[wid])

    f = pl.kernel(
        body,
        out_type=jax.ShapeDtypeStruct((NW, rows, 128), jnp.float32),
        mesh=_sc_mesh(),
        name=f"segcount_s{s_pad}_k{k}",
        compiler_params=pltpu.CompilerParams(needs_layout_passes=False),
        scratch_types=[
            pltpu.VMEM((k, CH), jnp.int32),
            pltpu.VMEM((rows, 128), jnp.float32),
        ],
    )
    return f(dst3)


def _cnt_reduce_call(parts):
    _, rows, d = parts.shape

    def body(a_ref, o_ref):
        o_ref[...] = jnp.sum(a_ref[...], axis=0)

    out = pl.pallas_call(
        body, out_shape=jax.ShapeDtypeStruct((rows, d), jnp.float32),
    )(parts)
    return out.reshape(rows * d, 1)


def _seq(x, dep):
    x, _ = lax.optimization_barrier((x, dep))
    return x


def _prelu(v, a):
    return jnp.where(v > 0, v, a * v)


def _mean(parts_ref, cnt_ref):
    s = parts_ref[0] + parts_ref[1]
    cnt = cnt_ref[...]
    return s / jnp.maximum(cnt, 1.0)


def _linear_call(h, w, b):
    m = h.shape[0]
    dout = w.shape[1]

    def body(h_ref, w_ref, b_ref, o_ref):
        o_ref[...] = jnp.dot(h_ref[...], w_ref[...],
                             preferred_element_type=jnp.float32) + b_ref[...]

    return pl.pallas_call(
        body, out_shape=jax.ShapeDtypeStruct((m, dout), jnp.float32),
    )(h, w, b.reshape(1, dout))


def _e_fusion_call(e_parts, cnt_parts, w2, b2, w3, b3, ae, n_e):
    d2 = w2.shape[1]
    d3 = w3.shape[1]

    def body(ep, cp, w2r, b2r, w3r, b3r, ae_r, e_o, ec_o, en_o):
        e = _prelu(_mean(ep, cp), ae_r[0, 0])
        e_o[...] = e[:n_e]
        ec_o[...] = (jnp.dot(e, w2r[...], preferred_element_type=jnp.float32)
                     + b2r[...])[:n_e]
        en_o[...] = (jnp.dot(e, w3r[...], preferred_element_type=jnp.float32)
                     + b3r[...])[:n_e]

    return pl.pallas_call(
        body,
        out_shape=[
            jax.ShapeDtypeStruct((n_e, e_parts.shape[2]), jnp.float32),
            jax.ShapeDtypeStruct((n_e, d2), jnp.float32),
            jax.ShapeDtypeStruct((n_e, d3), jnp.float32),
        ],
    )(e_parts, cnt_parts, w2, b2.reshape(1, d2), w3, b3.reshape(1, d3),
      ae.reshape(1, 1))


def _c_fusion_call(c_parts, cnt_parts, w4, b4, ac, n_c):
    din = w4.shape[0]
    d4 = w4.shape[1]

    def body(cparts, cnt, w4r, b4r, ac_r, c_o, cn_o):
        c = _prelu(_mean(cparts, cnt), ac_r[0, 0])[:, :din]
        c_o[...] = c[:n_c]
        cn_o[...] = (jnp.dot(c, w4r[...], preferred_element_type=jnp.float32)
                     + b4r[...])[:n_c]

    return pl.pallas_call(
        body,
        out_shape=[
            jax.ShapeDtypeStruct((n_c, din), jnp.float32),
            jax.ShapeDtypeStruct((n_c, d4), jnp.float32),
        ],
    )(c_parts, cnt_parts, w4, b4.reshape(1, d4), ac.reshape(1, 1))


def _n_fusion_call(nfe_parts, cnfe, nfc_parts, cnfc, an, alpha, n_n):
    d = nfe_parts.shape[2]

    def body(ep, ec, cp, cc, an_r, al_r, h_o):
        n = _mean(ep, ec) + _mean(cp, cc)
        n = _prelu(n, an_r[0, 0])
        h = _prelu(n, al_r[0, 0])
        h_o[...] = h[:n_n]

    return pl.pallas_call(
        body,
        out_shape=jax.ShapeDtypeStruct((n_n, d), jnp.float32),
    )(nfe_parts, cnfe, nfc_parts, cnfc, an.reshape(1, 1), alpha.reshape(1, 1))


def kernel(x, hyperedge_index, hyperedge_component_index, node_component_index,
           num_nodes, num_edges, num_components, params, alpha_act):
    n_n = x.shape[0]
    n_e = hyperedge_component_index.shape[1]
    n_c = N_COMP_STATIC

    s_e = _round_up(n_e + 1, NS * ZB)
    s_c = _round_up(n_c + 1, NS * ZB)
    s_n = _round_up(n_n + 1, NS * ZB)

    hei = hyperedge_index
    hci = hyperedge_component_index
    nci = node_component_index

    se_src, se_dst = _prep_indices(hei[0], hei[1], n_e, s_e, n_n)
    sn_src, sn_dst = _prep_indices(hei[1], hei[0], n_n, s_n, n_e)
    sc_src, sc_dst = _prep_indices(hci[0], hci[1], n_c, s_c, n_e)
    ncs_src, ncs_dst = _prep_indices(nci[1], nci[0], n_n, s_n, n_c)

    cnt_e = _cnt_reduce_call(_segcount_call(se_dst, s_e))
    cnt_ne = _cnt_reduce_call(_segcount_call(sn_dst, s_n))
    cnt_c = _cnt_reduce_call(_segcount_call(sc_dst, s_c))
    cnt_nc = _cnt_reduce_call(_segcount_call(ncs_dst, s_n))

    h = x
    e = c = None
    for p in params:
        w2p = jnp.pad(p['W2'], ((0, 0), (0, 128 - p['W2'].shape[1])))
        b2p = jnp.pad(p['b2'], (0, 128 - p['b2'].shape[0]))
        xe = _linear_call(h, p['W1'], p['b1'])
        e_parts = _segsum_call(xe, se_src, se_dst, s_e)
        e, ec, en = _e_fusion_call(e_parts, cnt_e, w2p, b2p,
                                   p['W3'], p['b3'], p['ae'], n_e)
        c_parts = _segsum_call(ec, sc_src, sc_dst, s_c)
        c, cn = _c_fusion_call(c_parts, cnt_c, p['W4'], p['b4'], p['ac'], n_c)
        nfe_parts = _segsum_call(en, sn_src, sn_dst, s_n)
        nfc_parts = _segsum_call(cn, ncs_src, ncs_dst, s_n)
        h = _n_fusion_call(nfe_parts, cnt_ne, nfc_parts, cnt_nc,
                           p['an'], alpha_act, n_n)
    return (h, e, c)


# speedup vs baseline: 4.1994x; 1.0115x over previous
import jax
import jax.numpy as jnp
from jax import lax
from jax.experimental import pallas as pl
from jax.experimental.pallas import tpu as pltpu
from jax.experimental.pallas import tpu_sc as plsc

N_COMP_STATIC = 1000

NC = 2
NS = 16
NW = NC * NS
CH = 128
G = 8
ZB = 16


def _round_up(n, m):
    return ((n + m - 1) // m) * m


def _prep_indices(src, dst, s, s_pad, r):
    n = src.shape[0]
    n_pad = _round_up(n, NW * CH * G)
    pad = n_pad - n
    if pad:
        ar = jnp.arange(pad, dtype=jnp.int32)
        src = jnp.concatenate([src, ar % r])
        dst = jnp.concatenate([dst, s + ar % (s_pad - s)])
    k = n_pad // (NW * CH)
    return src.reshape(NW, k, CH), dst.reshape(NW, k, CH)


def _fill_const(ref, rows, d, value):
    def body(i, carry):
        for j in range(d // 16):
            ref[i, pl.ds(j * 16, 16)] = jnp.full((16,), value, jnp.float32)
        return carry
    lax.fori_loop(0, rows, body, 0)


def _sc_mesh():
    return plsc.VectorSubcoreMesh(core_axis_name="c", subcore_axis_name="s",
                                  num_cores=NC, num_subcores=NS)


def _segsum_call(table, src3, dst3, s_pad, with_counts=False):
    _, d = table.shape
    k = src3.shape[1]
    rpt = s_pad // NS
    crows = s_pad // 128
    kg = k // G

    def body(table_h, src_h, dst_h, *rest):
        if with_counts:
            (out_h, cnt_h, src_v, dst_v, zb_v, acc_sh, cnt_v, r0, r1,
             g0, g1) = rest
        else:
            out_h, src_v, dst_v, zb_v, acc_sh, r0, r1, g0, g1 = rest
        cid = lax.axis_index("c")
        sid = lax.axis_index("s")
        wid = sid * NC + cid
        base = sid * rpt
        _fill_const(zb_v, ZB, d, 0.0)

        def zacc(b, carry):
            pltpu.sync_copy(zb_v, acc_sh.at[pl.ds(base + b * ZB, ZB)])
            return carry
        lax.fori_loop(0, rpt // ZB, zacc, 0)
        if with_counts:
            _fill_const(cnt_v, crows, 128, 0.0)
        plsc.subcore_barrier()
        ones = jnp.ones((16,), jnp.float32)

        def group(g, carry):
            pltpu.sync_copy(src_h.at[wid, pl.ds(g * G, G)], src_v)
            pltpu.sync_copy(dst_h.at[wid, pl.ds(g * G, G)], dst_v)
            if with_counts:
                def h_outer(j, c2):
                    def h_inner(t, c3):
                        ii = dst_v[j, pl.ds(t * 16, 16)]
                        row = lax.shift_right_logical(ii, 7)
                        col = lax.bitwise_and(ii, 127)
                        plsc.addupdate_scatter(cnt_v, [row, col], ones)
                        return c3
                    lax.fori_loop(0, CH // 16, h_inner, 0)
                    return c2
                lax.fori_loop(0, G, h_outer, 0)

            def step(jb, carry2):
                j = jb * 2
                pltpu.async_copy(table_h.at[src_v.at[j]], r0, g0)
                pltpu.async_copy(table_h.at[src_v.at[j + 1]], r1, g1)
                pltpu.make_async_copy(table_h.at[src_v.at[j]], r0, g0).wait()
                pltpu.sync_copy(r0, acc_sh.at[dst_v.at[j]], add=True)
                pltpu.make_async_copy(table_h.at[src_v.at[j + 1]], r1,
                                      g1).wait()
                pltpu.sync_copy(r1, acc_sh.at[dst_v.at[j + 1]], add=True)
                return carry2
            lax.fori_loop(0, G // 2, step, 0)
            return carry
        lax.fori_loop(0, kg, group, 0)
        if with_counts:
            pltpu.sync_copy(cnt_v, cnt_h.at[wid])
        plsc.subcore_barrier()
        pltpu.sync_copy(acc_sh.at[pl.ds(base, rpt)],
                        out_h.at[cid, pl.ds(base, rpt)])

    out_type = [jax.ShapeDtypeStruct((NC, s_pad, d), jnp.float32)]
    cnt_scratch = []
    if with_counts:
        out_type.append(jax.ShapeDtypeStruct((NW, crows, 128), jnp.float32))
        cnt_scratch = [pltpu.VMEM((crows, 128), jnp.float32)]
    f = pl.kernel(
        body,
        out_type=out_type,
        mesh=_sc_mesh(),
        name=f"segsum_s{s_pad}_k{k}_c{int(with_counts)}",
        compiler_params=pltpu.CompilerParams(needs_layout_passes=False),
        scratch_types=[
            pltpu.VMEM((G, CH), jnp.int32),
            pltpu.VMEM((G, CH), jnp.int32),
            pltpu.VMEM((ZB, d), jnp.float32),
            pltpu.VMEM_SHARED((s_pad, d), jnp.float32),
        ] + cnt_scratch
          + [pltpu.VMEM((CH, d), jnp.float32) for _ in range(2)]
          + [pltpu.SemaphoreType.DMA for _ in range(2)],
    )
    res = f(table, src3, dst3)
    return res if with_counts else res[0]


def _cnt_reduce_call(parts):
    _, rows, d = parts.shape

    def body(a_ref, o_ref):
        o_ref[...] = jnp.sum(a_ref[...], axis=0)

    out = pl.pallas_call(
        body, out_shape=jax.ShapeDtypeStruct((rows, d), jnp.float32),
    )(parts)
    return out.reshape(rows * d, 1)


def _prelu(v, a):
    return jnp.where(v > 0, v, a * v)


def _mean(parts_ref, cnt_ref):
    s = parts_ref[0] + parts_ref[1]
    cnt = cnt_ref[...]
    return s / jnp.maximum(cnt, 1.0)


def _linear_call(h, w, b):
    m = h.shape[0]
    dout = w.shape[1]

    def body(h_ref, w_ref, b_ref, o_ref):
        o_ref[...] = jnp.dot(h_ref[...], w_ref[...],
                             preferred_element_type=jnp.float32) + b_ref[...]

    return pl.pallas_call(
        body, out_shape=jax.ShapeDtypeStruct((m, dout), jnp.float32),
    )(h, w, b.reshape(1, dout))


def _e_fusion_call(e_parts, cnt_parts, w2, b2, w3, b3, ae, n_e):
    d2 = w2.shape[1]
    d3 = w3.shape[1]

    def body(ep, cp, w2r, b2r, w3r, b3r, ae_r, e_o, ec_o, en_o):
        e = _prelu(_mean(ep, cp), ae_r[0, 0])
        e_o[...] = e[:n_e]
        ec_o[...] = (jnp.dot(e, w2r[...], preferred_element_type=jnp.float32)
                     + b2r[...])[:n_e]
        en_o[...] = (jnp.dot(e, w3r[...], preferred_element_type=jnp.float32)
                     + b3r[...])[:n_e]

    return pl.pallas_call(
        body,
        out_shape=[
            jax.ShapeDtypeStruct((n_e, e_parts.shape[2]), jnp.float32),
            jax.ShapeDtypeStruct((n_e, d2), jnp.float32),
            jax.ShapeDtypeStruct((n_e, d3), jnp.float32),
        ],
    )(e_parts, cnt_parts, w2, b2.reshape(1, d2), w3, b3.reshape(1, d3),
      ae.reshape(1, 1))


def _c_fusion_call(c_parts, cnt_parts, w4, b4, ac, n_c):
    din = w4.shape[0]
    d4 = w4.shape[1]

    def body(cparts, cnt, w4r, b4r, ac_r, c_o, cn_o):
        c = _prelu(_mean(cparts, cnt), ac_r[0, 0])[:, :din]
        c_o[...] = c[:n_c]
        cn_o[...] = (jnp.dot(c, w4r[...], preferred_element_type=jnp.float32)
                     + b4r[...])[:n_c]

    return pl.pallas_call(
        body,
        out_shape=[
            jax.ShapeDtypeStruct((n_c, din), jnp.float32),
            jax.ShapeDtypeStruct((n_c, d4), jnp.float32),
        ],
    )(c_parts, cnt_parts, w4, b4.reshape(1, d4), ac.reshape(1, 1))


def _n_fusion_call(nfe_parts, cnfe, nfc_parts, cnfc, an, alpha, n_n):
    d = nfe_parts.shape[2]

    def body(ep, ec, cp, cc, an_r, al_r, h_o):
        n = _mean(ep, ec) + _mean(cp, cc)
        n = _prelu(n, an_r[0, 0])
        h = _prelu(n, al_r[0, 0])
        h_o[...] = h[:n_n]

    return pl.pallas_call(
        body,
        out_shape=jax.ShapeDtypeStruct((n_n, d), jnp.float32),
    )(nfe_parts, cnfe, nfc_parts, cnfc, an.reshape(1, 1), alpha.reshape(1, 1))


def kernel(x, hyperedge_index, hyperedge_component_index, node_component_index,
           num_nodes, num_edges, num_components, params, alpha_act):
    n_n = x.shape[0]
    n_e = hyperedge_component_index.shape[1]
    n_c = N_COMP_STATIC

    s_e = _round_up(n_e + 1, NS * ZB)
    s_c = _round_up(n_c + NS * ZB, NS * ZB)
    s_n = _round_up(n_n + 1, NS * ZB)

    hei = hyperedge_index
    hci = hyperedge_component_index
    nci = node_component_index

    se_src, se_dst = _prep_indices(hei[0], hei[1], n_e, s_e, n_n)
    sn_src, sn_dst = _prep_indices(hei[1], hei[0], n_n, s_n, n_e)
    sc_src, sc_dst = _prep_indices(hci[0], hci[1], n_c, s_c, n_e)
    ncs_src, ncs_dst = _prep_indices(nci[1], nci[0], n_n, s_n, n_c)

    h = x
    e = c = None
    cnt_e = cnt_ne = cnt_c = cnt_nc = None
    for li, p in enumerate(params):
        first = li == 0
        w2p = jnp.pad(p['W2'], ((0, 0), (0, 128 - p['W2'].shape[1])))
        b2p = jnp.pad(p['b2'], (0, 128 - p['b2'].shape[0]))
        xe = _linear_call(h, p['W1'], p['b1'])
        if first:
            e_parts, ce = _segsum_call(xe, se_src, se_dst, s_e, True)
            cnt_e = _cnt_reduce_call(ce)
        else:
            e_parts = _segsum_call(xe, se_src, se_dst, s_e)
        e, ec, en = _e_fusion_call(e_parts, cnt_e, w2p, b2p,
                                   p['W3'], p['b3'], p['ae'], n_e)
        if first:
            c_parts, cc = _segsum_call(ec, sc_src, sc_dst, s_c, True)
            cnt_c = _cnt_reduce_call(cc)
        else:
            c_parts = _segsum_call(ec, sc_src, sc_dst, s_c)
        c, cn = _c_fusion_call(c_parts, cnt_c, p['W4'], p['b4'], p['ac'], n_c)
        if first:
            nfe_parts, cne = _segsum_call(en, sn_src, sn_dst, s_n, True)
            cnt_ne = _cnt_reduce_call(cne)
            nfc_parts, cnc = _segsum_call(cn, ncs_src, ncs_dst, s_n, True)
            cnt_nc = _cnt_reduce_call(cnc)
        else:
            nfe_parts = _segsum_call(en, sn_src, sn_dst, s_n)
            nfc_parts = _segsum_call(cn, ncs_src, ncs_dst, s_n)
        h = _n_fusion_call(nfe_parts, cnt_ne, nfc_parts, cnt_nc,
                           p['an'], alpha_act, n_n)
    return (h, e, c)
